# Initial kernel scaffold; baseline (speedup 1.0000x reference)
import functools

import jax
import jax.numpy as jnp
from jax import lax
from jax.experimental import pallas as pl
from jax.experimental.pallas import tpu as pltpu
from jax.experimental.pallas import tpu_sc as plsc

_SHAPE = 100000
_SCALE = 1.0
_DIM = 32
_DX = (_SHAPE - 1) // 2

_B, _S = 4096, 200
_N = _B * _S

_INFO = plsc.get_sparse_core_info()
_NC, _NS, _L = _INFO.num_cores, _INFO.num_subcores, _INFO.num_lanes
_NW = _NC * _NS
_PER_W = _N // _NW
_C = 128
_NCHUNK = _PER_W // _C


def _body(pos_hbm, emb_hbm, out_hbm, pos_v, idx_v, rows_v, sem):
    wid = lax.axis_index("s") * _NC + lax.axis_index("c")
    w_base = wid * _PER_W

    def chunk(c, _):
        base = w_base + c * _C
        pltpu.sync_copy(pos_hbm.at[pl.ds(base, _C)], pos_v)
        for j in range(_C // _L):
            p = pos_v[pl.ds(j * _L, _L)]
            x = (p * (_DX / _SCALE) + _DX).astype(jnp.int32)
            idx_v[pl.ds(j * _L, _L)] = jnp.clip(x, 0, _SHAPE - 1)
        pltpu.async_copy(emb_hbm.at[idx_v], rows_v, sem).wait()
        pltpu.sync_copy(rows_v, out_hbm.at[pl.ds(base, _C)])
        return 0

    lax.fori_loop(0, _NCHUNK, chunk, 0)


@jax.jit
def _xembed(pos_flat, embedding):
    mesh = plsc.VectorSubcoreMesh(core_axis_name="c", subcore_axis_name="s")
    return pl.kernel(
        _body,
        mesh=mesh,
        out_type=jax.ShapeDtypeStruct((_N, _DIM), jnp.float32),
        scratch_types=[
            pltpu.VMEM((_C,), jnp.float32),
            pltpu.VMEM((_C,), jnp.int32),
            pltpu.VMEM((_C, _DIM), jnp.float32),
            pltpu.SemaphoreType.DMA,
        ],
    )(pos_flat, embedding)


def kernel(pos, embedding):
    out = _xembed(pos.reshape(_N), embedding)
    return out.reshape(_B, _S, _DIM)


# speedup vs baseline: 1.5051x; 1.5051x over previous
import functools

import jax
import jax.numpy as jnp
from jax import lax
from jax.experimental import pallas as pl
from jax.experimental.pallas import tpu as pltpu
from jax.experimental.pallas import tpu_sc as plsc

_SHAPE = 100000
_SCALE = 1.0
_DIM = 32
_DX = (_SHAPE - 1) // 2

_B, _S = 4096, 200
_N = _B * _S

_INFO = plsc.get_sparse_core_info()
_NC, _NS, _L = _INFO.num_cores, _INFO.num_subcores, _INFO.num_lanes
_NW = _NC * _NS
_PER_W = _N // _NW
_C = 128
_NCHUNK = _PER_W // _C


def _body(pos_hbm, emb_hbm, out_hbm, pos_v, idx_v, rows_v, sem):
    wid = lax.axis_index("s") * _NC + lax.axis_index("c")
    w_base = wid * _PER_W

    def chunk(c, _):
        base = w_base + c * _C
        pltpu.sync_copy(pos_hbm.at[pl.ds(base, _C)], pos_v)
        for j in range(_C // _L):
            p = pos_v[pl.ds(j * _L, _L)]
            x = (p * (_DX / _SCALE) + _DX).astype(jnp.int32)
            idx_v[pl.ds(j * _L, _L)] = jnp.clip(x, 0, _SHAPE - 1)
        pltpu.async_copy(emb_hbm.at[idx_v], rows_v, sem).wait()
        pltpu.sync_copy(rows_v, out_hbm.at[pl.ds(base, _C)])
        return 0

    lax.fori_loop(0, _NCHUNK, chunk, 0)


@jax.jit
def _xembed(pos_flat, embedding):
    mesh = plsc.VectorSubcoreMesh(core_axis_name="c", subcore_axis_name="s")
    return pl.kernel(
        _body,
        mesh=mesh,
        compiler_params=pltpu.CompilerParams(use_tc_tiling_on_sc=False),
        out_type=jax.ShapeDtypeStruct((_N, _DIM), jnp.float32),
        scratch_types=[
            pltpu.VMEM((_C,), jnp.float32),
            pltpu.VMEM((_C,), jnp.int32),
            pltpu.VMEM((_C, _DIM), jnp.float32),
            pltpu.SemaphoreType.DMA,
        ],
    )(pos_flat, embedding)


def kernel(pos, embedding):
    out = _xembed(pos.reshape(_N), embedding)
    return out.reshape(_B, _S, _DIM)


# speedup vs baseline: 1.5163x; 1.0075x over previous
import jax
import jax.numpy as jnp
from jax import lax
from jax.experimental import pallas as pl
from jax.experimental.pallas import tpu as pltpu
from jax.experimental.pallas import tpu_sc as plsc

_SHAPE = 100000
_SCALE = 1.0
_DIM = 32
_DX = (_SHAPE - 1) // 2

_B, _S = 4096, 200
_N = _B * _S

_INFO = plsc.get_sparse_core_info()
_NC, _NS, _L = _INFO.num_cores, _INFO.num_subcores, _INFO.num_lanes
_NW = _NC * _NS
_PER_W = _N // _NW
_C = 512
_NBUF = 4
_NCHUNK = _PER_W // _C


def _body(pos_hbm, emb_hbm, out_hbm, pos_v, idx_v, rows_v, gsem, wsem):
    wid = lax.axis_index("s") * _NC + lax.axis_index("c")
    w_base = wid * _PER_W

    pltpu.sync_copy(pos_hbm.at[pl.ds(w_base, _PER_W)], pos_v)

    def cvt(j, _):
        p = pos_v[pl.ds(j * _L, _L)]
        x = (p * (_DX / _SCALE) + _DX).astype(jnp.int32)
        idx_v[pl.ds(j * _L, _L)] = jnp.clip(x, 0, _SHAPE - 1)
        return 0

    lax.fori_loop(0, _PER_W // _L, cvt, 0)

    def g_start(c):
        b = c % _NBUF
        pltpu.async_copy(
            emb_hbm.at[idx_v.at[pl.ds(c * _C, _C)]], rows_v.at[b], gsem.at[b]
        )

    def g_wait(c):
        b = c % _NBUF
        pltpu.make_async_copy(
            emb_hbm.at[idx_v.at[pl.ds(c * _C, _C)]], rows_v.at[b], gsem.at[b]
        ).wait()

    def w_start(c):
        b = c % _NBUF
        pltpu.async_copy(
            rows_v.at[b], out_hbm.at[pl.ds(w_base + c * _C, _C)], wsem.at[b]
        )

    def w_wait(c):
        b = c % _NBUF
        pltpu.make_async_copy(
            rows_v.at[b], out_hbm.at[pl.ds(w_base + c * _C, _C)], wsem.at[b]
        ).wait()

    for c in range(_NBUF - 1):
        g_start(c)
    for c in range(_NCHUNK):
        nxt = c + _NBUF - 1
        if nxt < _NCHUNK:
            if nxt >= _NBUF:
                w_wait(nxt - _NBUF)
            g_start(nxt)
        g_wait(c)
        w_start(c)
    for c in range(_NCHUNK - _NBUF, _NCHUNK):
        w_wait(c)


@jax.jit
def _xembed(pos_flat, embedding):
    mesh = plsc.VectorSubcoreMesh(core_axis_name="c", subcore_axis_name="s")
    return pl.kernel(
        _body,
        mesh=mesh,
        compiler_params=pltpu.CompilerParams(use_tc_tiling_on_sc=False),
        out_type=jax.ShapeDtypeStruct((_N, _DIM), jnp.float32),
        scratch_types=[
            pltpu.VMEM((_PER_W,), jnp.float32),
            pltpu.VMEM((_PER_W,), jnp.int32),
            pltpu.VMEM((_NBUF, _C, _DIM), jnp.float32),
            pltpu.SemaphoreType.DMA((_NBUF,)),
            pltpu.SemaphoreType.DMA((_NBUF,)),
        ],
    )(pos_flat, embedding)


def kernel(pos, embedding):
    out = _xembed(pos.reshape(_N), embedding)
    return out.reshape(_B, _S, _DIM)


# speedup vs baseline: 2.5954x; 1.7117x over previous
import jax
import jax.numpy as jnp
from jax import lax
from jax.experimental import pallas as pl
from jax.experimental.pallas import tpu as pltpu
from jax.experimental.pallas import tpu_sc as plsc

_SHAPE = 100000
_SCALE = 1.0
_DIM = 32
_DX = (_SHAPE - 1) // 2

_B, _S = 4096, 200
_N = _B * _S

_INFO = plsc.get_sparse_core_info()
_NC, _NS, _L = _INFO.num_cores, _INFO.num_subcores, _INFO.num_lanes
_HALF = _SHAPE // _NC
_STAGE = _HALF // _NS
_PER_T = _N // _NS
_C = 256
_NBUF = 2
_NSTEP = _PER_T // (_C * _NBUF)
_DUMP = 512


def _body(pos_hbm, emb_hbm, out_hbm, shared, pos_v, lidx_v, opos_v, rows_v, gsem, ssem):
    cid = lax.axis_index("c")
    sid = lax.axis_index("s")
    lo = cid * _HALF
    t_base = sid * _PER_T

    if True:
        pltpu.sync_copy(
            emb_hbm.at[pl.ds(lo + sid * _STAGE, _STAGE)],
            shared.at[pl.ds(sid * _STAGE, _STAGE)],
        )
        plsc.subcore_barrier()

        lane = lax.iota(jnp.int32, _L)

        def step(st, _):
            base = t_base + st * (_C * _NBUF)
            for b in range(_NBUF):
                cb = base + b * _C
                pltpu.sync_copy(pos_hbm.at[pl.ds(cb, _C)], pos_v.at[b])
                for j in range(_C // _L):
                    p = pos_v[b, pl.ds(j * _L, _L)]
                    x = (p * (_DX / _SCALE) + _DX).astype(jnp.int32)
                    x = jnp.clip(x, 0, _SHAPE - 1)
                    loc = x - lo
                    m = (loc >= 0) & (loc < _HALF)
                    gpos = cb + j * _L + lane
                    lidx_v[b, pl.ds(j * _L, _L)] = jnp.where(m, loc, 0)
                    opos_v[b, pl.ds(j * _L, _L)] = jnp.where(
                        m, gpos, _N + (gpos & (_DUMP - 1))
                    )
                pltpu.async_copy(shared.at[lidx_v.at[b]], rows_v.at[b], gsem.at[b])
            for b in range(_NBUF):
                pltpu.make_async_copy(
                    shared.at[lidx_v.at[b]], rows_v.at[b], gsem.at[b]
                ).wait()
                pltpu.async_copy(rows_v.at[b], out_hbm.at[opos_v.at[b]], ssem.at[b])
            for b in range(_NBUF):
                pltpu.make_async_copy(
                    rows_v.at[b], out_hbm.at[opos_v.at[b]], ssem.at[b]
                ).wait()
            return 0

        lax.fori_loop(0, _NSTEP, step, 0)


@jax.jit
def _xembed(pos_flat, embedding):
    mesh = plsc.VectorSubcoreMesh(core_axis_name="c", subcore_axis_name="s")
    return pl.kernel(
        _body,
        mesh=mesh,
        compiler_params=pltpu.CompilerParams(use_tc_tiling_on_sc=False),
        out_type=jax.ShapeDtypeStruct((_N + _DUMP, _DIM), jnp.float32),
        scratch_types=[
            pltpu.VMEM_SHARED((_HALF, _DIM), jnp.float32),
            pltpu.VMEM((_NBUF, _C), jnp.float32),
            pltpu.VMEM((_NBUF, _C), jnp.int32),
            pltpu.VMEM((_NBUF, _C), jnp.int32),
            pltpu.VMEM((_NBUF, _C, _DIM), jnp.float32),
            pltpu.SemaphoreType.DMA((_NBUF,)),
            pltpu.SemaphoreType.DMA((_NBUF,)),
        ],
    )(pos_flat, embedding)


def kernel(pos, embedding):
    out = _xembed(pos.reshape(_N), embedding)
    return out[:_N].reshape(_B, _S, _DIM)


# speedup vs baseline: 4.1200x; 1.5874x over previous
import jax
import jax.numpy as jnp
from jax import lax
from jax.experimental import pallas as pl
from jax.experimental.pallas import tpu as pltpu
from jax.experimental.pallas import tpu_sc as plsc

_SHAPE = 100000
_SCALE = 1.0
_DIM = 32
_DX = (_SHAPE - 1) // 2

_B, _S = 4096, 200
_N = _B * _S

_INFO = plsc.get_sparse_core_info()
_NC, _NS, _L = _INFO.num_cores, _INFO.num_subcores, _INFO.num_lanes
_HALF = _SHAPE // _NC
_STAGE = _HALF // _NS
_PER_T = _N // _NS
_SEG = 6400
_NSEG = _PER_T // _SEG
_CH = 640
_NCH = _SEG // _CH
_C = 128
_CAP = _SEG + 2 * _C


def _body(pos_hbm, emb_hbm, out_hbm, shared, pos_v, cidx, opos, rows0, rows1, sem0, sem1):
    cid = lax.axis_index("c")
    sid = lax.axis_index("s")
    lo = cid * _HALF
    t_base = sid * _PER_T
    lane = lax.iota(jnp.int32, _L)

    pltpu.sync_copy(
        emb_hbm.at[pl.ds(lo + sid * _STAGE, _STAGE)],
        shared.at[pl.ds(sid * _STAGE, _STAGE)],
    )
    plsc.subcore_barrier()

    for seg in range(_NSEG):
        seg_base = t_base + seg * _SEG

        def chunk(ch, off):
            pltpu.sync_copy(pos_hbm.at[pl.ds(seg_base + ch * _CH, _CH)], pos_v)

            def grp(g, off):
                p = pos_v[pl.ds(g * _L, _L)]
                x = (p * (_DX / _SCALE) + _DX).astype(jnp.int32)
                x = jnp.clip(x, 0, _SHAPE - 1)
                loc = x - lo
                m = (loc >= 0) & (loc < _HALF)
                gpos = seg_base + ch * _CH + g * _L + lane
                pre = plsc.cumsum(m.astype(jnp.int32))
                tgt = jnp.where(m, off + pre - 1, _CAP - _L + lane)
                plsc.store_scatter(cidx, [tgt], loc)
                plsc.store_scatter(opos, [tgt], gpos)
                return off + plsc.all_reduce_population_count(m)[0]

            return lax.fori_loop(0, _CH // _L, grp, off)

        off = lax.fori_loop(0, _NCH, chunk, jnp.int32(0))

        @pl.when(off > 0)
        def _():
            last_l = cidx[pl.ds(off - 1, _L)][0]
            last_o = opos[pl.ds(off - 1, _L)][0]
            for q in range(_C // _L):
                cidx[pl.ds(off + q * _L, _L)] = jnp.full((_L,), last_l, jnp.int32)
                opos[pl.ds(off + q * _L, _L)] = jnp.full((_L,), last_o, jnp.int32)

        plsc.subcore_barrier()

        nb = (off + _C - 1) // _C

        def bat(b, _):
            def fire(rb, sem):
                @pl.when(b >= 2)
                def _():
                    pltpu.make_async_copy(
                        rb, out_hbm.at[opos.at[pl.ds(0, _C)]], sem
                    ).wait()

                pltpu.sync_copy(shared.at[cidx.at[pl.ds(b * _C, _C)]], rb)
                pltpu.async_copy(rb, out_hbm.at[opos.at[pl.ds(b * _C, _C)]], sem)

            even = (b & 1) == 0

            @pl.when(even)
            def _():
                fire(rows0, sem0)

            @pl.when(jnp.logical_not(even))
            def _():
                fire(rows1, sem1)

            return 0

        lax.fori_loop(0, nb, bat, 0)

        for back in (1, 2):
            @pl.when(nb >= back)
            def _():
                @pl.when(((nb - back) & 1) == 0)
                def _():
                    pltpu.make_async_copy(
                        rows0, out_hbm.at[opos.at[pl.ds(0, _C)]], sem0
                    ).wait()

                @pl.when(((nb - back) & 1) == 1)
                def _():
                    pltpu.make_async_copy(
                        rows1, out_hbm.at[opos.at[pl.ds(0, _C)]], sem1
                    ).wait()


@jax.jit
def _xembed(pos_flat, embedding):
    mesh = plsc.VectorSubcoreMesh(core_axis_name="c", subcore_axis_name="s")
    return pl.kernel(
        _body,
        mesh=mesh,
        compiler_params=pltpu.CompilerParams(
            use_tc_tiling_on_sc=False, needs_layout_passes=False
        ),
        out_type=jax.ShapeDtypeStruct((_N, _DIM), jnp.float32),
        scratch_types=[
            pltpu.VMEM_SHARED((_HALF, _DIM), jnp.float32),
            pltpu.VMEM((_CH,), jnp.float32),
            pltpu.VMEM((_CAP,), jnp.int32),
            pltpu.VMEM((_CAP,), jnp.int32),
            pltpu.VMEM((_C, _DIM), jnp.float32),
            pltpu.VMEM((_C, _DIM), jnp.float32),
            pltpu.SemaphoreType.DMA,
            pltpu.SemaphoreType.DMA,
        ],
    )(pos_flat, embedding)


def kernel(pos, embedding):
    out = _xembed(pos.reshape(_N), embedding)
    return out.reshape(_B, _S, _DIM)


# speedup vs baseline: 4.3255x; 1.0499x over previous
import jax
import jax.numpy as jnp
from jax import lax
from jax.experimental import pallas as pl
from jax.experimental.pallas import tpu as pltpu
from jax.experimental.pallas import tpu_sc as plsc

_SHAPE = 100000
_SCALE = 1.0
_DIM = 32
_DX = (_SHAPE - 1) // 2

_B, _S = 4096, 200
_N = _B * _S

_INFO = plsc.get_sparse_core_info()
_NC, _NS, _L = _INFO.num_cores, _INFO.num_subcores, _INFO.num_lanes
_HALF = _SHAPE // _NC
_STAGE = _HALF // _NS
_PER_T = _N // _NS
_SEG = 5120
_NSEG = _PER_T // _SEG
_CH = 640
_NCH = _SEG // _CH
_UNR = 4
_C = 256
_CAP = _SEG + 2 * _C


def _body(pos_hbm, emb_hbm, out_hbm, shared, pos_v, cidx, opos, rows0, rows1, psem, sem0, sem1):
    cid = lax.axis_index("c")
    sid = lax.axis_index("s")
    lo = cid * _HALF
    t_base = sid * _PER_T
    lane = lax.iota(jnp.int32, _L)

    pltpu.sync_copy(
        emb_hbm.at[pl.ds(lo + sid * _STAGE, _STAGE)],
        shared.at[pl.ds(sid * _STAGE, _STAGE)],
    )
    plsc.subcore_barrier()

    def pos_load(ch_glob, buf):
        pltpu.async_copy(
            pos_hbm.at[pl.ds(t_base + ch_glob * _CH, _CH)], pos_v.at[buf], psem
        )

    def pos_wait(buf):
        pltpu.make_async_copy(
            pos_hbm.at[pl.ds(t_base, _CH)], pos_v.at[buf], psem
        ).wait()

    pos_load(0, 0)

    for seg in range(_NSEG):
        seg_base = t_base + seg * _SEG

        def chunk(ch, off):
            ch_glob = seg * _NCH + ch
            buf = ch_glob & 1
            pos_wait(buf)

            @pl.when(ch_glob + 1 < _NSEG * _NCH)
            def _():
                pos_load(ch_glob + 1, 1 - buf)

            def grp(gq, off):
                for u in range(_UNR):
                    g = gq * _UNR + u
                    p = pos_v[buf, pl.ds(g * _L, _L)]
                    x = (p * (_DX / _SCALE) + _DX).astype(jnp.int32)
                    x = jnp.clip(x, 0, _SHAPE - 1)
                    loc = x - lo
                    m = (loc >= 0) & (loc < _HALF)
                    gpos = seg_base + ch * _CH + g * _L + lane
                    pre = plsc.cumsum(m.astype(jnp.int32))
                    tgt = jnp.where(m, off + pre - 1, _CAP - _L + lane)
                    plsc.store_scatter(cidx, [tgt], loc)
                    plsc.store_scatter(opos, [tgt], gpos)
                    off = off + plsc.all_reduce_population_count(m)[0]
                return off

            return lax.fori_loop(0, _CH // (_UNR * _L), grp, off)

        off = lax.fori_loop(0, _NCH, chunk, jnp.int32(0))

        @pl.when(off > 0)
        def _():
            last_l = cidx[pl.ds(off - 1, _L)][0]
            last_o = opos[pl.ds(off - 1, _L)][0]
            for q in range(_C // _L):
                cidx[pl.ds(off + q * _L, _L)] = jnp.full((_L,), last_l, jnp.int32)
                opos[pl.ds(off + q * _L, _L)] = jnp.full((_L,), last_o, jnp.int32)

        plsc.subcore_barrier()

        nb = (off + _C - 1) // _C

        def bat(b, _):
            def fire(rb, sem):
                @pl.when(b >= 2)
                def _():
                    pltpu.make_async_copy(
                        rb, out_hbm.at[opos.at[pl.ds(0, _C)]], sem
                    ).wait()

                pltpu.sync_copy(shared.at[cidx.at[pl.ds(b * _C, _C)]], rb)
                pltpu.async_copy(rb, out_hbm.at[opos.at[pl.ds(b * _C, _C)]], sem)

            even = (b & 1) == 0

            @pl.when(even)
            def _():
                fire(rows0, sem0)

            @pl.when(jnp.logical_not(even))
            def _():
                fire(rows1, sem1)

            return 0

        lax.fori_loop(0, nb, bat, 0)

        for back in (1, 2):
            @pl.when(nb >= back)
            def _():
                @pl.when(((nb - back) & 1) == 0)
                def _():
                    pltpu.make_async_copy(
                        rows0, out_hbm.at[opos.at[pl.ds(0, _C)]], sem0
                    ).wait()

                @pl.when(((nb - back) & 1) == 1)
                def _():
                    pltpu.make_async_copy(
                        rows1, out_hbm.at[opos.at[pl.ds(0, _C)]], sem1
                    ).wait()


@jax.jit
def _xembed(pos_flat, embedding):
    mesh = plsc.VectorSubcoreMesh(core_axis_name="c", subcore_axis_name="s")
    return pl.kernel(
        _body,
        mesh=mesh,
        compiler_params=pltpu.CompilerParams(
            use_tc_tiling_on_sc=False, needs_layout_passes=False
        ),
        out_type=jax.ShapeDtypeStruct((_N, _DIM), jnp.float32),
        scratch_types=[
            pltpu.VMEM_SHARED((_HALF, _DIM), jnp.float32),
            pltpu.VMEM((2, _CH), jnp.float32),
            pltpu.VMEM((_CAP,), jnp.int32),
            pltpu.VMEM((_CAP,), jnp.int32),
            pltpu.VMEM((_C, _DIM), jnp.float32),
            pltpu.VMEM((_C, _DIM), jnp.float32),
            pltpu.SemaphoreType.DMA,
            pltpu.SemaphoreType.DMA,
            pltpu.SemaphoreType.DMA,
        ],
    )(pos_flat, embedding)


def kernel(pos, embedding):
    out = _xembed(pos.reshape(_N), embedding)
    return out.reshape(_B, _S, _DIM)
